# baseline (device time: 56574 ns/iter reference)
import jax
import jax.numpy as jnp
from jax import lax
from jax.experimental import pallas as pl
from jax.experimental.pallas import tpu as pltpu

N_DEV = 4
SQ = 1024
SKV_LOC = 1024
HQ = 8
DH = 128
D = 1024
SCALE = 0.08838834764831843
G = 32
B1 = 896
NB1 = SQ - B1
NS = G + NB1
BR = 128
BW = 384
NBLK = SQ // BR

F32 = jnp.float32
BF16 = jnp.bfloat16


def _band_masks():
    r = lax.broadcasted_iota(jnp.int32, (BR, BW), 0)
    c = lax.broadcasted_iota(jnp.int32, (BR, BW), 1)
    m0 = ((jnp.abs(r - c) <= 128) | (c < 32)) & (r >= 32)
    m1 = ((c >= r) & (c <= r + 256)) | (c < 32)
    mg = (c >= r) & (c <= r + 256)
    m7 = c >= r + 128
    return m0, m1, mg, m7


def kernel(x, Wq, K_ext, V_ext, Wo):
    K2 = K_ext.reshape(SKV_LOC, HQ * DH).astype(BF16)
    V2 = V_ext.reshape(SKV_LOC, HQ * DH).astype(BF16)

    def body(x_ref, wq_ref, k_ref, v_ref, wo_ref, out_ref,
             acc_send, l_send, bc_msg, bcast_acc, ctx_buf, sc_buf,
             strip_acc, strip_l,
             bc_send_sems, bc_recv_sems, fwd_send_sems,
             sc_send_sems, sc_recv_sems,
             strip_send_sems, strip_recv_sems):
        my = lax.axis_index("i")

        def rcopy(src, dst, ssem, rsem, dev):
            return pltpu.make_async_remote_copy(
                src_ref=src, dst_ref=dst, send_sem=ssem, recv_sem=rsem,
                device_id=(dev,), device_id_type=pl.DeviceIdType.MESH)

        def wait_recv(dst, rsem):
            rcopy(dst, dst, rsem, rsem, 0).wait_recv()

        def wait_send(src, ssem):
            rcopy(src, src, ssem, ssem, 0).wait_send()

        def mm(a, b):
            return jnp.dot(a, b, preferred_element_type=F32)

        def mmT(a, b):
            return lax.dot_general(a, b, (((1,), (1,)), ((), ())),
                                   preferred_element_type=F32)

        barrier_sem = pltpu.get_barrier_semaphore()
        for off in (1, 2, 3):
            pl.semaphore_signal(barrier_sem, inc=1,
                                device_id=((my + off) % N_DEV,),
                                device_id_type=pl.DeviceIdType.MESH)
        pl.semaphore_wait(barrier_sem, N_DEV - 1)

        rows_g = pl.ds(0, G)
        rows_b1 = pl.ds(B1, NB1)
        rows_sg = pl.ds(0, G)
        rows_sb1 = pl.ds(G, NB1)
        wqb = wq_ref[...].astype(BF16)

        @pl.when(my == 0)
        def _():
            q = mm(x_ref[0].astype(BF16), wqb)
            q = (q * SCALE).astype(BF16)
            m0, m1, mg, m7 = _band_masks()
            cglob = lax.broadcasted_iota(jnp.int32, (SQ - 2 * BR, BR), 1) < 32
            for h in range(HQ):
                sl = slice(h * DH, (h + 1) * DH)
                kh = k_ref[:, sl]
                vh = v_ref[:, sl]
                qh = q[:, sl]
                w_g = jnp.exp(mmT(qh[0:G], kh))
                l_gl = jnp.sum(w_g, axis=1, keepdims=True)
                acc_gl = mm(w_g.astype(BF16), vh)
                acc_blocks, l_blocks = [], []
                for b in range(NBLK):
                    w0 = min(max(0, BR * b - BR), SKV_LOC - BW)
                    mask = {0: m0, 1: m1, NBLK - 1: m7}.get(b, mg)
                    s_b = mmT(qh[BR * b:BR * b + BR], kh[w0:w0 + BW])
                    w_b = jnp.where(mask, jnp.exp(s_b), 0.0)
                    lb = jnp.sum(w_b, axis=1, keepdims=True)
                    accb = mm(w_b.astype(BF16), vh[w0:w0 + BW])
                    if b == 0:
                        accb, lb = accb[G:BR], lb[G:BR]
                    acc_blocks.append(accb)
                    l_blocks.append(lb)
                s_s = mmT(qh[2 * BR:SQ], kh[0:BR])
                w_s = jnp.where(cglob, jnp.exp(s_s), 0.0)
                acc_tail = (jnp.concatenate(acc_blocks[2:], axis=0)
                            + mm(w_s.astype(BF16), vh[0:BR]))
                l_tail = (jnp.concatenate(l_blocks[2:], axis=0)
                          + jnp.sum(w_s, axis=1, keepdims=True))
                acc_h = jnp.concatenate(
                    [acc_gl, acc_blocks[0], acc_blocks[1], acc_tail], axis=0)
                l_h = jnp.concatenate(
                    [l_gl, l_blocks[0], l_blocks[1], l_tail], axis=0)
                ctx_mid = (acc_h[G:B1] / l_h[G:B1]).astype(BF16)
                bc_msg[h] = jnp.concatenate(
                    [acc_h[0:G].astype(BF16), ctx_mid,
                     acc_h[B1:SQ].astype(BF16)], axis=0)
                l_send[:, h:h + 1] = l_h
                ctx_buf[G:B1, sl] = ctx_mid
                dsts = ((0, 1), (1, 3), (2, 2)) if h == HQ - 1 \
                    else ((0, 1), (1, 3))
                for d_i, dst in dsts:
                    rcopy(bc_msg.at[h], bcast_acc.at[h],
                          bc_send_sems.at[d_i, h], bc_recv_sems.at[h],
                          dst).start()
            for s in (1, 2, 3):
                wait_recv(strip_acc.at[s, rows_sg, :],
                          strip_recv_sems.at[s, 0])
                wait_recv(strip_l.at[s, rows_sg, :],
                          strip_recv_sems.at[s, 2])
            wait_recv(strip_acc.at[1, rows_sb1, :], strip_recv_sems.at[1, 1])
            wait_recv(strip_l.at[1, rows_sb1, :], strip_recv_sems.at[1, 3])
            for h in range(HQ):
                sl = slice(h * DH, (h + 1) * DH)
                accg = bc_msg[h, 0:G, :].astype(F32)
                lg = l_send[rows_g, h:h + 1]
                for s in (1, 2, 3):
                    accg = accg + strip_acc[s, rows_sg, sl].astype(F32)
                    lg = lg + strip_l[s, rows_sg, h:h + 1]
                ctx_g = (accg / lg).astype(BF16)
                accb1 = (bc_msg[h, B1:SQ, :].astype(F32)
                         + strip_acc[1, rows_sb1, sl].astype(F32))
                lb1 = l_send[rows_b1, h:h + 1] + strip_l[1, rows_sb1, h:h + 1]
                ctx_b1 = (accb1 / lb1).astype(BF16)
                sc_buf[rows_sg, sl] = ctx_g
                sc_buf[rows_sb1, sl] = ctx_b1
                ctx_buf[rows_g, sl] = ctx_g
                ctx_buf[rows_b1, sl] = ctx_b1
            for d_i, dst in enumerate((1, 2, 3)):
                rcopy(sc_buf, sc_buf, sc_send_sems.at[d_i],
                      sc_recv_sems.at[0], dst).start()

        def strip_compute_send(src_id, with_b1):
            q_g = (mm(x_ref[0, 0:G, :].astype(BF16), wqb)
                   * SCALE).astype(BF16)
            if with_b1:
                q_b1 = (mm(x_ref[0, B1:SQ, :].astype(BF16), wqb)
                        * SCALE).astype(BF16)
            for h in range(HQ):
                sl = slice(h * DH, (h + 1) * DH)
                kh = k_ref[:, sl]
                vh = v_ref[:, sl]
                w_g = jnp.exp(mmT(q_g[:, sl], kh))
                l_send[rows_g, h:h + 1] = jnp.sum(w_g, axis=1, keepdims=True)
                acc_send[rows_g, sl] = mm(w_g.astype(BF16), vh).astype(BF16)
                if with_b1:
                    r = lax.broadcasted_iota(jnp.int32, (NB1, BR), 0)
                    c = lax.broadcasted_iota(jnp.int32, (NB1, BR), 1)
                    s_b = mmT(q_b1[:, sl], kh[0:BR])
                    w_b = jnp.where(c <= r, jnp.exp(s_b), 0.0)
                    l_send[rows_b1, h:h + 1] = jnp.sum(w_b, axis=1,
                                                       keepdims=True)
                    acc_send[rows_b1, sl] = mm(w_b.astype(BF16),
                                               vh[0:BR]).astype(BF16)
            rcopy(acc_send.at[rows_g, :], strip_acc.at[src_id, rows_sg, :],
                  strip_send_sems.at[0], strip_recv_sems.at[src_id, 0],
                  0).start()
            rcopy(l_send.at[rows_g, :], strip_l.at[src_id, rows_sg, :],
                  strip_send_sems.at[2], strip_recv_sems.at[src_id, 2],
                  0).start()
            if with_b1:
                rcopy(acc_send.at[rows_b1, :],
                      strip_acc.at[src_id, rows_sb1, :],
                      strip_send_sems.at[1], strip_recv_sems.at[src_id, 1],
                      0).start()
                rcopy(l_send.at[rows_b1, :], strip_l.at[src_id, rows_sb1, :],
                      strip_send_sems.at[3], strip_recv_sems.at[src_id, 3],
                      0).start()

        def recv_side(role, with_b1, fwd_parity):
            strip_compute_send(role, with_b1)
            for h in range(HQ):
                sl = slice(h * DH, (h + 1) * DH)
                wait_recv(bcast_acc.at[h], bc_recv_sems.at[h])
                if (fwd_parity is not None and h % 2 == fwd_parity
                        and h != HQ - 1):
                    rcopy(bcast_acc.at[h], bcast_acc.at[h],
                          fwd_send_sems.at[h], bc_recv_sems.at[h], 2).start()
                ctx_buf[:, sl] = bcast_acc[h]
            wait_recv(sc_buf, sc_recv_sems.at[0])
            ctx_buf[rows_g, :] = sc_buf[rows_sg, :]
            ctx_buf[rows_b1, :] = sc_buf[rows_sb1, :]

        @pl.when(my == 1)
        def _():
            recv_side(1, True, 0)

        @pl.when(my == 2)
        def _():
            recv_side(2, False, None)

        @pl.when(my == 3)
        def _():
            recv_side(3, False, 1)

        out_ref[0] = mm(ctx_buf[...], wo_ref[...].astype(BF16))

        @pl.when(my == 0)
        def _():
            for h in range(HQ):
                for d_i in (0, 1):
                    wait_send(bc_msg.at[h], bc_send_sems.at[d_i, h])
            wait_send(bc_msg.at[HQ - 1], bc_send_sems.at[2, HQ - 1])
            for d_i in range(3):
                wait_send(sc_buf, sc_send_sems.at[d_i])

        def drain_strips(with_b1):
            wait_send(acc_send.at[rows_g, :], strip_send_sems.at[0])
            wait_send(l_send.at[rows_g, :], strip_send_sems.at[2])
            if with_b1:
                wait_send(acc_send.at[rows_b1, :], strip_send_sems.at[1])
                wait_send(l_send.at[rows_b1, :], strip_send_sems.at[3])

        @pl.when(my == 1)
        def _():
            drain_strips(True)
            for h in (0, 2, 4, 6):
                wait_send(bcast_acc.at[h], fwd_send_sems.at[h])

        @pl.when(my == 2)
        def _():
            drain_strips(False)

        @pl.when(my == 3)
        def _():
            drain_strips(False)
            for h in (1, 3, 5):
                wait_send(bcast_acc.at[h], fwd_send_sems.at[h])

        def exit_barrier(second_barrier):
            for off in (1, 2, 3):
                pl.semaphore_signal(second_barrier, inc=1,
                                    device_id=((my + off) % N_DEV,),
                                    device_id_type=pl.DeviceIdType.MESH)
            pl.semaphore_wait(second_barrier, N_DEV - 1)

        pl.run_scoped(exit_barrier,
                      second_barrier=pltpu.SemaphoreType.REGULAR)

    out = pl.pallas_call(
        body,
        out_shape=jax.ShapeDtypeStruct((1, SQ, D), jnp.float32),
        in_specs=[pl.BlockSpec(memory_space=pltpu.VMEM)] * 5,
        out_specs=pl.BlockSpec(memory_space=pltpu.VMEM),
        scratch_shapes=[
            pltpu.VMEM((SQ, D), BF16),
            pltpu.VMEM((SQ, HQ), F32),
            pltpu.VMEM((HQ, SQ, DH), BF16),
            pltpu.VMEM((HQ, SQ, DH), BF16),
            pltpu.VMEM((SQ, D), BF16),
            pltpu.VMEM((NS, D), BF16),
            pltpu.VMEM((N_DEV, NS, D), BF16),
            pltpu.VMEM((N_DEV, NS, HQ), F32),
            pltpu.SemaphoreType.DMA((3, HQ)),
            pltpu.SemaphoreType.DMA((HQ,)),
            pltpu.SemaphoreType.DMA((HQ,)),
            pltpu.SemaphoreType.DMA((3,)),
            pltpu.SemaphoreType.DMA((1,)),
            pltpu.SemaphoreType.DMA((4,)),
            pltpu.SemaphoreType.DMA((N_DEV, 4)),
        ],
        compiler_params=pltpu.CompilerParams(collective_id=0),
    )(x, Wq, K2, V2, Wo)
    return out


# device time: 54098 ns/iter; 1.0458x vs baseline; 1.0458x over previous
import jax
import jax.numpy as jnp
from jax import lax
from jax.experimental import pallas as pl
from jax.experimental.pallas import tpu as pltpu

N_DEV = 4
SQ = 1024
SKV_LOC = 1024
HQ = 8
DH = 128
D = 1024
SCALE = 0.08838834764831843
G = 32
B1 = 896
NB1 = SQ - B1
NS = G + NB1
BR = 128
BW = 384
NBLK = SQ // BR

F32 = jnp.float32
BF16 = jnp.bfloat16


def _band_masks():
    r = lax.broadcasted_iota(jnp.int32, (BR, BW), 0)
    c = lax.broadcasted_iota(jnp.int32, (BR, BW), 1)
    m0 = ((jnp.abs(r - c) <= 128) | (c < 32)) & (r >= 32)
    m1 = ((c >= r) & (c <= r + 256)) | (c < 32)
    mg = (c >= r) & (c <= r + 256)
    m7 = c >= r + 128
    return m0, m1, mg, m7


def kernel(x, Wq, K_ext, V_ext, Wo):
    K2 = K_ext.reshape(SKV_LOC, HQ * DH).astype(BF16)
    V2 = V_ext.reshape(SKV_LOC, HQ * DH).astype(BF16)

    def body(x_ref, wq_ref, k_ref, v_ref, wo_ref, out_ref,
             acc_send, l_send, bc_msg, bcast_acc, ctx_buf, sc_buf,
             strip_acc, strip_l,
             bc_send_sems, bc_recv_sems, fwd_send_sems,
             sc_send_sems, sc_recv_sems,
             strip_send_sems, strip_recv_sems):
        my = lax.axis_index("i")

        def rcopy(src, dst, ssem, rsem, dev):
            return pltpu.make_async_remote_copy(
                src_ref=src, dst_ref=dst, send_sem=ssem, recv_sem=rsem,
                device_id=(dev,), device_id_type=pl.DeviceIdType.MESH)

        def wait_recv(dst, rsem):
            rcopy(dst, dst, rsem, rsem, 0).wait_recv()

        def wait_send(src, ssem):
            rcopy(src, src, ssem, ssem, 0).wait_send()

        def mm(a, b):
            return jnp.dot(a, b, preferred_element_type=F32)

        def mmT(a, b):
            return lax.dot_general(a, b, (((1,), (1,)), ((), ())),
                                   preferred_element_type=F32)

        barrier_sem = pltpu.get_barrier_semaphore()
        for off in (1, 2, 3):
            pl.semaphore_signal(barrier_sem, inc=1,
                                device_id=((my + off) % N_DEV,),
                                device_id_type=pl.DeviceIdType.MESH)
        pl.semaphore_wait(barrier_sem, N_DEV - 1)

        rows_g = pl.ds(0, G)
        rows_b1 = pl.ds(B1, NB1)
        rows_sg = pl.ds(0, G)
        rows_sb1 = pl.ds(G, NB1)
        wqb = wq_ref[...].astype(BF16)

        @pl.when(my == 0)
        def _():
            q = mm(x_ref[0].astype(BF16), wqb)
            q = (q * SCALE).astype(BF16)
            m0, m1, mg, m7 = _band_masks()
            cglob = lax.broadcasted_iota(jnp.int32, (SQ - 2 * BR, BR), 1) < 32
            for h in range(HQ):
                sl = slice(h * DH, (h + 1) * DH)
                kh = k_ref[:, sl]
                vh = v_ref[:, sl]
                qh = q[:, sl]
                w_g = jnp.exp(mmT(qh[0:G], kh))
                l_gl = jnp.sum(w_g, axis=1, keepdims=True)
                acc_gl = mm(w_g.astype(BF16), vh)
                acc_blocks, l_blocks = [], []
                for b in range(NBLK):
                    w0 = min(max(0, BR * b - BR), SKV_LOC - BW)
                    mask = {0: m0, 1: m1, NBLK - 1: m7}.get(b, mg)
                    s_b = mmT(qh[BR * b:BR * b + BR], kh[w0:w0 + BW])
                    w_b = jnp.where(mask, jnp.exp(s_b), 0.0)
                    lb = jnp.sum(w_b, axis=1, keepdims=True)
                    accb = mm(w_b.astype(BF16), vh[w0:w0 + BW])
                    if b == 0:
                        accb, lb = accb[G:BR], lb[G:BR]
                    acc_blocks.append(accb)
                    l_blocks.append(lb)
                s_s = mmT(qh[2 * BR:SQ], kh[0:BR])
                w_s = jnp.where(cglob, jnp.exp(s_s), 0.0)
                acc_tail = (jnp.concatenate(acc_blocks[2:], axis=0)
                            + mm(w_s.astype(BF16), vh[0:BR]))
                l_tail = (jnp.concatenate(l_blocks[2:], axis=0)
                          + jnp.sum(w_s, axis=1, keepdims=True))
                acc_h = jnp.concatenate(
                    [acc_gl, acc_blocks[0], acc_blocks[1], acc_tail], axis=0)
                l_h = jnp.concatenate(
                    [l_gl, l_blocks[0], l_blocks[1], l_tail], axis=0)
                ctx_mid = (acc_h[G:B1] / l_h[G:B1]).astype(BF16)
                bc_msg[h] = jnp.concatenate(
                    [acc_h[0:G].astype(BF16), ctx_mid,
                     acc_h[B1:SQ].astype(BF16)], axis=0)
                l_send[:, h:h + 1] = l_h
                ctx_buf[G:B1, sl] = ctx_mid
                for d_i, dst in ((0, 1), (1, 3)):
                    rcopy(bc_msg.at[h], bcast_acc.at[h],
                          bc_send_sems.at[d_i, h], bc_recv_sems.at[h],
                          dst).start()
            for s in (1, 2, 3):
                wait_recv(strip_acc.at[s, rows_sg, :],
                          strip_recv_sems.at[s, 0])
                wait_recv(strip_l.at[s, rows_sg, :],
                          strip_recv_sems.at[s, 2])
            wait_recv(strip_acc.at[1, rows_sb1, :], strip_recv_sems.at[1, 1])
            wait_recv(strip_l.at[1, rows_sb1, :], strip_recv_sems.at[1, 3])
            for h in range(HQ):
                sl = slice(h * DH, (h + 1) * DH)
                accg = bc_msg[h, 0:G, :].astype(F32)
                lg = l_send[rows_g, h:h + 1]
                for s in (1, 2, 3):
                    accg = accg + strip_acc[s, rows_sg, sl].astype(F32)
                    lg = lg + strip_l[s, rows_sg, h:h + 1]
                ctx_g = (accg / lg).astype(BF16)
                accb1 = (bc_msg[h, B1:SQ, :].astype(F32)
                         + strip_acc[1, rows_sb1, sl].astype(F32))
                lb1 = l_send[rows_b1, h:h + 1] + strip_l[1, rows_sb1, h:h + 1]
                ctx_b1 = (accb1 / lb1).astype(BF16)
                sc_buf[rows_sg, sl] = ctx_g
                sc_buf[rows_sb1, sl] = ctx_b1
                ctx_buf[rows_g, sl] = ctx_g
                ctx_buf[rows_b1, sl] = ctx_b1
            for d_i, dst in enumerate((1, 2, 3)):
                rcopy(sc_buf, sc_buf, sc_send_sems.at[d_i],
                      sc_recv_sems.at[0], dst).start()

        def strip_compute_send(src_id, with_b1):
            q_g = (mm(x_ref[0, 0:G, :].astype(BF16), wqb)
                   * SCALE).astype(BF16)
            if with_b1:
                q_b1 = (mm(x_ref[0, B1:SQ, :].astype(BF16), wqb)
                        * SCALE).astype(BF16)
            for h in range(HQ):
                sl = slice(h * DH, (h + 1) * DH)
                kh = k_ref[:, sl]
                vh = v_ref[:, sl]
                w_g = jnp.exp(mmT(q_g[:, sl], kh))
                l_send[rows_g, h:h + 1] = jnp.sum(w_g, axis=1, keepdims=True)
                acc_send[rows_g, sl] = mm(w_g.astype(BF16), vh).astype(BF16)
                if with_b1:
                    r = lax.broadcasted_iota(jnp.int32, (NB1, BR), 0)
                    c = lax.broadcasted_iota(jnp.int32, (NB1, BR), 1)
                    s_b = mmT(q_b1[:, sl], kh[0:BR])
                    w_b = jnp.where(c <= r, jnp.exp(s_b), 0.0)
                    l_send[rows_b1, h:h + 1] = jnp.sum(w_b, axis=1,
                                                       keepdims=True)
                    acc_send[rows_b1, sl] = mm(w_b.astype(BF16),
                                               vh[0:BR]).astype(BF16)
            rcopy(acc_send.at[rows_g, :], strip_acc.at[src_id, rows_sg, :],
                  strip_send_sems.at[0], strip_recv_sems.at[src_id, 0],
                  0).start()
            rcopy(l_send.at[rows_g, :], strip_l.at[src_id, rows_sg, :],
                  strip_send_sems.at[2], strip_recv_sems.at[src_id, 2],
                  0).start()
            if with_b1:
                rcopy(acc_send.at[rows_b1, :],
                      strip_acc.at[src_id, rows_sb1, :],
                      strip_send_sems.at[1], strip_recv_sems.at[src_id, 1],
                      0).start()
                rcopy(l_send.at[rows_b1, :], strip_l.at[src_id, rows_sb1, :],
                      strip_send_sems.at[3], strip_recv_sems.at[src_id, 3],
                      0).start()

        def recv_side(role, with_b1, fwd_parity):
            strip_compute_send(role, with_b1)
            for h in range(HQ):
                sl = slice(h * DH, (h + 1) * DH)
                wait_recv(bcast_acc.at[h], bc_recv_sems.at[h])
                if fwd_parity is not None and h % 2 == fwd_parity:
                    rcopy(bcast_acc.at[h], bcast_acc.at[h],
                          fwd_send_sems.at[h], bc_recv_sems.at[h], 2).start()
                ctx_buf[:, sl] = bcast_acc[h]
            wait_recv(sc_buf, sc_recv_sems.at[0])
            ctx_buf[rows_g, :] = sc_buf[rows_sg, :]
            ctx_buf[rows_b1, :] = sc_buf[rows_sb1, :]

        @pl.when(my == 1)
        def _():
            recv_side(1, True, 0)

        @pl.when(my == 2)
        def _():
            recv_side(2, False, None)

        @pl.when(my == 3)
        def _():
            recv_side(3, False, 1)

        out_ref[0] = mm(ctx_buf[...], wo_ref[...].astype(BF16))

        @pl.when(my == 0)
        def _():
            for h in range(HQ):
                for d_i in (0, 1):
                    wait_send(bc_msg.at[h], bc_send_sems.at[d_i, h])
            for d_i in range(3):
                wait_send(sc_buf, sc_send_sems.at[d_i])

        def drain_strips(with_b1):
            wait_send(acc_send.at[rows_g, :], strip_send_sems.at[0])
            wait_send(l_send.at[rows_g, :], strip_send_sems.at[2])
            if with_b1:
                wait_send(acc_send.at[rows_b1, :], strip_send_sems.at[1])
                wait_send(l_send.at[rows_b1, :], strip_send_sems.at[3])

        @pl.when(my == 1)
        def _():
            drain_strips(True)
            for h in (0, 2, 4, 6):
                wait_send(bcast_acc.at[h], fwd_send_sems.at[h])

        @pl.when(my == 2)
        def _():
            drain_strips(False)

        @pl.when(my == 3)
        def _():
            drain_strips(False)
            for h in (1, 3, 5, 7):
                wait_send(bcast_acc.at[h], fwd_send_sems.at[h])

        def exit_barrier(second_barrier):
            for off in (1, 2, 3):
                pl.semaphore_signal(second_barrier, inc=1,
                                    device_id=((my + off) % N_DEV,),
                                    device_id_type=pl.DeviceIdType.MESH)
            pl.semaphore_wait(second_barrier, N_DEV - 1)

        pl.run_scoped(exit_barrier,
                      second_barrier=pltpu.SemaphoreType.REGULAR)

    out = pl.pallas_call(
        body,
        out_shape=jax.ShapeDtypeStruct((1, SQ, D), jnp.float32),
        in_specs=[pl.BlockSpec(memory_space=pltpu.VMEM)] * 5,
        out_specs=pl.BlockSpec(memory_space=pltpu.VMEM),
        scratch_shapes=[
            pltpu.VMEM((SQ, D), BF16),
            pltpu.VMEM((SQ, HQ), F32),
            pltpu.VMEM((HQ, SQ, DH), BF16),
            pltpu.VMEM((HQ, SQ, DH), BF16),
            pltpu.VMEM((SQ, D), BF16),
            pltpu.VMEM((NS, D), BF16),
            pltpu.VMEM((N_DEV, NS, D), BF16),
            pltpu.VMEM((N_DEV, NS, HQ), F32),
            pltpu.SemaphoreType.DMA((2, HQ)),
            pltpu.SemaphoreType.DMA((HQ,)),
            pltpu.SemaphoreType.DMA((HQ,)),
            pltpu.SemaphoreType.DMA((3,)),
            pltpu.SemaphoreType.DMA((1,)),
            pltpu.SemaphoreType.DMA((4,)),
            pltpu.SemaphoreType.DMA((N_DEV, 4)),
        ],
        compiler_params=pltpu.CompilerParams(collective_id=0),
    )(x, Wq, K2, V2, Wo)
    return out
